# baseline (speedup 1.0000x reference)
import functools

import jax
import jax.numpy as jnp
import numpy as np
from jax import lax
from jax.experimental import pallas as pl
from jax.experimental.pallas import tpu as pltpu
from jax.experimental.pallas import tpu_sc as plsc

_MAX_NORM = 1.0

_B = 4096
_SUBROWS = 407
_NW = 32
_RW = _B // _NW
_R = 8
_NCH = _RW // _R
_CW = _R * _SUBROWS
_NQ = 416
_RAW_W = 131

_FIELDS = (
    (12, 4, 0, 0),
    (48, 4, 4096, 12),
    (24, 2, 8192, 60),
    (12, 4, 8232, 84),
    (12, 4, 9432, 96),
    (23, 1, 13528, 108),
)


def _qmaps():
    qoff, qn, qscale, qbase = [], [], [], []
    for n, k, tb, so in _FIELDS:
        for i in range(n):
            for j in range(k):
                qoff.append(so * _RW + i)
                qn.append(n)
                qscale.append(k)
                qbase.append(tb + j if k > 1 else tb + 2 * i)
    pad = _NQ - len(qoff)
    mk = lambda x: np.asarray(x + [0] * pad, dtype=np.int32)
    return mk(qoff), mk(qn), mk(qscale), mk(qbase)


_QMAPS = _qmaps()


def _normalize_tables_tc(*tables):
    def body(*refs):
        n = len(refs) // 2
        for src, dst in zip(refs[:n], refs[n:]):
            x = src[...]
            nrm = jnp.sqrt(jnp.sum(x * x, axis=-1, keepdims=True))
            scale = jnp.where(nrm > _MAX_NORM,
                              _MAX_NORM / jnp.maximum(nrm, 1e-12), 1.0)
            dst[...] = x * scale

    out_shapes = [jax.ShapeDtypeStruct(t.shape, t.dtype) for t in tables]
    return pl.pallas_call(body, out_shape=out_shapes)(*tables)


def _sc_gather(u, states, qmaps):
    mesh = plsc.VectorSubcoreMesh(core_axis_name="c", subcore_axis_name="s")

    raw_len = _RAW_W * _RW

    @functools.partial(
        pl.kernel,
        mesh=mesh,
        compiler_params=pltpu.CompilerParams(use_tc_tiling_on_sc=False,
                                             needs_layout_passes=False),
        out_type=jax.ShapeDtypeStruct((_B * _SUBROWS, 16), jnp.float32),
        scratch_types=[
            pltpu.VMEM((raw_len,), jnp.int32),
            pltpu.VMEM((4, _NQ), jnp.int32),
            pltpu.VMEM((2, _CW + 16), jnp.int32),
            pltpu.VMEM((2 * _CW, 16), jnp.float32),
            pltpu.SemaphoreType.DMA,
            pltpu.SemaphoreType.DMA,
        ],
    )
    def k(u_hbm, p_h, m_h, t_h, a_h, i_h, f_h, qm_h, out_hbm,
          raw_v, qm_v, idx_v, buf_v, sem_g, sem_w):
        wid = lax.axis_index("s") * 2 + lax.axis_index("c")
        pltpu.sync_copy(qm_h, qm_v)
        for h, (n, _, _, so) in zip((p_h, m_h, t_h, a_h, i_h, f_h), _FIELDS):
            pltpu.sync_copy(h.at[wid], raw_v.at[pl.ds(so * _RW, n * _RW)])
        out_w0 = wid * (_NCH * _CW)
        lanes = lax.iota(jnp.int32, 16)

        def compute_idx(c, sel):
            r0 = c * _R
            selv = jnp.full((16,), sel, jnp.int32)

            def jstep(j, carry):
                qo = qm_v[0, pl.ds(j * 16, 16)]
                qn = qm_v[1, pl.ds(j * 16, 16)]
                qs = qm_v[2, pl.ds(j * 16, 16)]
                qb = qm_v[3, pl.ds(j * 16, 16)]
                addr = qo + r0 * qn
                qpos = lanes + j * 16
                mask = qpos < _SUBROWS
                for r in range(_R):
                    rawv = plsc.load_gather(raw_v, [addr])
                    flat = qs * rawv + qb
                    pos = qpos + r * _SUBROWS
                    plsc.store_scatter(idx_v, [selv, pos], flat, mask=mask)
                    addr = addr + qn
                return carry

            lax.fori_loop(0, _NQ // 16, jstep, 0)

        def fire_gather(c, sel):
            pltpu.async_copy(u_hbm.at[idx_v.at[sel, pl.ds(0, _CW)]],
                             buf_v.at[pl.ds(sel * _CW, _CW)], sem_g)

        compute_idx(0, 0)
        fire_gather(0, 0)

        def chunk(c, carry):
            cur = c % 2
            nxt = (c + 1) % 2

            @pl.when(c >= 1)
            def _():
                pltpu.make_async_copy(buf_v.at[pl.ds(nxt * _CW, _CW)],
                                      out_hbm.at[pl.ds(out_w0, _CW)],
                                      sem_w).wait()

            @pl.when(c + 1 < _NCH)
            def _():
                compute_idx(c + 1, nxt)
                fire_gather(c + 1, nxt)

            pltpu.make_async_copy(u_hbm.at[pl.ds(0, _CW)],
                                  buf_v.at[pl.ds(cur * _CW, _CW)],
                                  sem_g).wait()
            pltpu.async_copy(buf_v.at[pl.ds(cur * _CW, _CW)],
                             out_hbm.at[pl.ds(out_w0 + c * _CW, _CW)], sem_w)
            return carry

        lax.fori_loop(0, _NCH, chunk, 0)
        pltpu.make_async_copy(buf_v.at[pl.ds(0, _CW)],
                              out_hbm.at[pl.ds(out_w0, _CW)], sem_w).wait()

    return k(u, *states, qmaps)


def kernel(pokemon_state, move_state, type_state, ability_state, item_state,
           fieldeffect_state, pokemon_table, move_table, type_table,
           ability_table, item_table, fieldeffect_tables):
    def _n(t):
        nrm = jnp.sqrt(jnp.sum(t * t, axis=-1, keepdims=True))
        return t * jnp.where(nrm > _MAX_NORM,
                             _MAX_NORM / jnp.maximum(nrm, 1e-12), 1.0)
    pt, mt, tt, at_, it, ft = (
        _n(pokemon_table), _n(move_table), _n(type_table), _n(ability_table),
        _n(item_table), _n(fieldeffect_tables.reshape(46, 16)))

    u = jnp.concatenate([
        pt.reshape(-1, 16), mt.reshape(-1, 16), tt.reshape(-1, 16),
        at_.reshape(-1, 16), it.reshape(-1, 16), ft,
    ], axis=0)

    states = [s.reshape(_NW, _RW * n) for s, (n, _, _, _) in zip(
        (pokemon_state, move_state, type_state, ability_state, item_state,
         fieldeffect_state), _FIELDS)]
    qmaps = jnp.asarray(np.stack(_QMAPS))

    out = _sc_gather(u, states, qmaps)
    return out.reshape(_B, _SUBROWS * 16)


# speedup vs baseline: 1.3295x; 1.3295x over previous
import functools

import jax
import jax.numpy as jnp
from jax import lax
from jax.experimental import pallas as pl
from jax.experimental.pallas import tpu as pltpu
from jax.experimental.pallas import tpu_sc as plsc

_B = 4096
_SUBROWS = 407
_NW = 32
_RW = _B // _NW
_R = 8
_NCH = _RW // _R
_CW = _R * _SUBROWS
_NU = 13574

_FIELDS = (
    (12, 2, 0, 0, 1024, 32, 2),
    (48, 2, 4096, 48, 1024, 32, 2),
    (24, 1, 8192, 240, 20, 2, 1),
    (12, 2, 8232, 288, 300, 19, 1),
    (12, 2, 9432, 336, 1024, 32, 2),
    (23, 0, 13528, 384, 46, 3, 1),
)


def _sc_kernel(tables, states):
    mesh = plsc.VectorSubcoreMesh(core_axis_name="c", subcore_axis_name="s")

    state_scratch = [pltpu.VMEM((_RW, n), jnp.int32)
                     for n, _, _, _, _, _, _ in _FIELDS]
    norm_scratch = [pltpu.VMEM((nr, t.shape[1]), jnp.float32)
                    for (_, _, _, _, _, nr, _), t in zip(_FIELDS, tables)]

    @functools.partial(
        pl.kernel,
        mesh=mesh,
        compiler_params=pltpu.CompilerParams(use_tc_tiling_on_sc=False,
                                             needs_layout_passes=False),
        out_type=[jax.ShapeDtypeStruct((_B * _SUBROWS, 16), jnp.float32),
                  jax.ShapeDtypeStruct((2, _NU, 16), jnp.float32)],
        scratch_types=state_scratch + norm_scratch + [
            pltpu.VMEM((128, 16), jnp.float32),
            pltpu.VMEM((2, _CW), jnp.int32),
            pltpu.VMEM((_CW, 16), jnp.float32),
            pltpu.SemaphoreType.DMA,
        ],
    )
    def k(*refs):
        tabs = refs[0:6]
        sth = refs[6:12]
        out_hbm, u_hbm = refs[12], refs[13]
        stv = refs[14:20]
        nrm = refs[20:26]
        ustage = refs[26]
        idx_v = refs[27]
        buf_v = refs[28]
        sem_g = refs[29]

        core = lax.axis_index("c")
        sub = lax.axis_index("s")
        wid = sub * 2 + core
        lanes = lax.iota(jnp.int32, 16)

        for (_, _, ubase, _, vocab, nr, np_), tab, nv in zip(
                _FIELDS, tabs, nrm):
            w = tab.shape[1]
            nsub = w // 16
            lo0 = (sub * vocab) // 16
            for p in range(np_):
                lo = lo0 + p * nr
                pltpu.sync_copy(tab.at[pl.ds(lo, nr)], nv)

                def row(i, carry):
                    vecs = [nv[i, pl.ds(kk * 16, 16)] for kk in range(nsub)]
                    sq = vecs[0] * vecs[0]
                    for v in vecs[1:]:
                        sq = sq + v * v
                    ss = jnp.sum(sq)
                    bits = lax.bitcast_convert_type(ss, jnp.int32)
                    y = lax.bitcast_convert_type(
                        jnp.int32(0x5F3759DF) - (bits >> 1), jnp.float32)
                    for _ in range(3):
                        y = y * (1.5 - 0.5 * ss * y * y)
                    scale = jnp.where(ss > 1.0, y, 1.0)
                    for kk in range(nsub):
                        ustage[i * nsub + kk] = vecs[kk] * scale
                    return carry

                lax.fori_loop(0, nr, row, 0)
                pltpu.sync_copy(ustage.at[pl.ds(0, nr * nsub)],
                                u_hbm.at[core, pl.ds(ubase + lo * nsub,
                                                     nr * nsub)])
        plsc.subcore_barrier()

        for h, v in zip(sth, stv):
            pltpu.sync_copy(h.at[pl.ds(wid * _RW, _RW)], v)
        out_w0 = wid * (_NCH * _CW)

        def compute_idx(c, sel):
            r0 = c * _R
            selv = jnp.full((16,), sel, jnp.int32)
            for (n, lg, ubase, q0, _, _, _), v in zip(_FIELDS, stv):
                nvec = (n << lg) // 16 if (n << lg) % 16 == 0 \
                    else ((n << lg) + 15) // 16
                for j in range(nvec):
                    qpos = lanes + j * 16
                    partial = (j + 1) * 16 > (n << lg)
                    mask = qpos < (n << lg) if partial else None
                    slot = qpos >> lg if lg else qpos
                    if partial:
                        slot = jnp.minimum(slot, n - 1)
                    if lg:
                        add = jnp.int32(ubase) + (qpos & ((1 << lg) - 1))
                    else:
                        add = jnp.int32(ubase) + 2 * qpos
                    for r in range(_R):
                        rows = jnp.full((16,), r0 + r, jnp.int32)
                        rawv = plsc.load_gather(v, [rows, slot])
                        flat = (rawv << lg) + add if lg else rawv + add
                        pos = qpos + (r * _SUBROWS + q0)
                        plsc.store_scatter(idx_v, [selv, pos], flat,
                                           mask=mask)

        def fire_gather(c, sel):
            pltpu.async_copy(
                u_hbm.at[core].at[idx_v.at[sel, pl.ds(0, _CW)]],
                buf_v, sem_g)

        compute_idx(0, 0)
        fire_gather(0, 0)

        def chunk(c, carry):
            nxt = (c + 1) % 2

            @pl.when(c + 1 < _NCH)
            def _():
                compute_idx(c + 1, nxt)

            pltpu.make_async_copy(u_hbm.at[core].at[pl.ds(0, _CW)],
                                  buf_v, sem_g).wait()
            pltpu.sync_copy(buf_v,
                            out_hbm.at[pl.ds(out_w0 + c * _CW, _CW)])

            @pl.when(c + 1 < _NCH)
            def _():
                fire_gather(c + 1, nxt)
            return carry

        lax.fori_loop(0, _NCH, chunk, 0)

    return k(*tables, *states)


def kernel(pokemon_state, move_state, type_state, ability_state, item_state,
           fieldeffect_state, pokemon_table, move_table, type_table,
           ability_table, item_table, fieldeffect_tables):
    out, _ = _sc_kernel(
        (pokemon_table, move_table, type_table, ability_table, item_table,
         fieldeffect_tables.reshape(46, 16)),
        (pokemon_state, move_state, type_state, ability_state, item_state,
         fieldeffect_state))
    return out.reshape(_B, _SUBROWS * 16)
